# baseline (speedup 1.0000x reference)
import functools

import jax
import jax.numpy as jnp
from jax import lax
from jax.experimental import pallas as pl
from jax.experimental.pallas import tpu as pltpu
from jax.experimental.pallas import tpu_sc as plsc

_NC = 2
_NS = 16
_NW = _NC * _NS
_CHUNK = 504
_TC_BLK = 25000
_TC_ROWS = 500000


def _tc_copy(ent_ref, rel_ref, ent_out, rel_out):
    ent_out[...] = ent_ref[...]

    @pl.when(pl.program_id(0) == 0)
    def _():
        rel_out[...] = rel_ref[...]


def _sc_slice_copy(ent_hbm, ent_out, *, start, count):
    rows = (count // _NW) // _CHUNK * _CHUNK
    nchunks = rows // _CHUNK
    tail_base = start + rows * _NW
    tail = count - rows * _NW

    mesh = plsc.VectorSubcoreMesh(core_axis_name="c", subcore_axis_name="s")

    @functools.partial(
        pl.kernel,
        mesh=mesh,
        out_type=jax.ShapeDtypeStruct(ent_hbm.shape, ent_hbm.dtype),
        scratch_types=[
            pltpu.VMEM((_CHUNK, ent_hbm.shape[1]), ent_hbm.dtype),
            pltpu.VMEM((_CHUNK, ent_hbm.shape[1]), ent_hbm.dtype),
            pltpu.SemaphoreType.DMA,
            pltpu.SemaphoreType.DMA,
            pltpu.SemaphoreType.DMA,
            pltpu.SemaphoreType.DMA,
            pltpu.SemaphoreType.DMA,
        ],
    )
    def _body(src, dst, buf0, buf1, isem0, isem1, osem0, osem1, tsem):
        wid = lax.axis_index("s") * _NC + lax.axis_index("c")
        base = start + wid * rows
        bufs = (buf0, buf1)
        isems = (isem0, isem1)
        osems = (osem0, osem1)

        @pl.when(wid == 0)
        def _():
            if tail:
                pltpu.make_async_copy(
                    src.at[pl.ds(tail_base, tail)],
                    dst.at[pl.ds(tail_base, tail)],
                    tsem,
                ).start()

        out_cps = [None, None]
        for i in range(nchunks):
            b = i % 2
            if out_cps[b] is not None:
                out_cps[b].wait()
            lo = base + i * _CHUNK
            icp = pltpu.make_async_copy(src.at[pl.ds(lo, _CHUNK)], bufs[b], isems[b])
            icp.start()
            icp.wait()
            ocp = pltpu.make_async_copy(bufs[b], dst.at[pl.ds(lo, _CHUNK)], osems[b])
            ocp.start()
            out_cps[b] = ocp
        for cp in out_cps:
            if cp is not None:
                cp.wait()

        @pl.when(wid == 0)
        def _():
            if tail:
                pltpu.make_async_copy(
                    src.at[pl.ds(tail_base, tail)],
                    dst.at[pl.ds(tail_base, tail)],
                    tsem,
                ).wait()

    return _body(ent_hbm)


def kernel(x_dict, edge_index, entity_emb, rel_emb):
    del x_dict, edge_index
    n_ent, d_ent = entity_emb.shape
    n_rel, d_rel = rel_emb.shape

    sc_full = _sc_slice_copy(entity_emb, None, start=_TC_ROWS, count=n_ent - _TC_ROWS)

    ent_head, rel = pl.pallas_call(
        _tc_copy,
        grid=(_TC_ROWS // _TC_BLK,),
        in_specs=[
            pl.BlockSpec((_TC_BLK, d_ent), lambda i: (i, 0)),
            pl.BlockSpec((n_rel, d_rel), lambda i: (0, 0)),
        ],
        out_specs=[
            pl.BlockSpec((_TC_BLK, d_ent), lambda i: (i, 0)),
            pl.BlockSpec((n_rel, d_rel), lambda i: (0, 0)),
        ],
        out_shape=[
            jax.ShapeDtypeStruct((_TC_ROWS, d_ent), entity_emb.dtype),
            jax.ShapeDtypeStruct((n_rel, d_rel), rel_emb.dtype),
        ],
    )(entity_emb, rel_emb)

    ent = jnp.concatenate([ent_head, sc_full[_TC_ROWS:]], axis=0)
    return (ent, rel)


# speedup vs baseline: 2.5045x; 2.5045x over previous
import jax
import jax.numpy as jnp
from jax.experimental import pallas as pl
from jax.experimental.pallas import tpu as pltpu

_NSTREAM = 8
_CHUNK = 5000


def _tc_multi(ent_hbm, rel_hbm, ent_out, rel_out, bufs, isems, osems, rsem):
    n = ent_hbm.shape[0]
    srows = n // _NSTREAM
    nchunks = srows // _CHUNK

    pltpu.make_async_copy(rel_hbm, rel_out, rsem).start()

    out_cps = [[None, None] for _ in range(_NSTREAM)]
    in_cps = [None] * _NSTREAM
    for i in range(nchunks):
        b = i % 2
        for s in range(_NSTREAM):
            if out_cps[s][b] is not None:
                out_cps[s][b].wait()
            lo = s * srows + i * _CHUNK
            cp = pltpu.make_async_copy(
                ent_hbm.at[pl.ds(lo, _CHUNK)], bufs.at[s, b], isems.at[s, b]
            )
            cp.start()
            in_cps[s] = cp
        for s in range(_NSTREAM):
            in_cps[s].wait()
            lo = s * srows + i * _CHUNK
            cp = pltpu.make_async_copy(
                bufs.at[s, b], ent_out.at[pl.ds(lo, _CHUNK)], osems.at[s, b]
            )
            cp.start()
            out_cps[s][b] = cp
    for s in range(_NSTREAM):
        for cp in out_cps[s]:
            if cp is not None:
                cp.wait()
    pltpu.make_async_copy(rel_hbm, rel_out, rsem).wait()


def kernel(x_dict, edge_index, entity_emb, rel_emb):
    del x_dict, edge_index
    n_ent, d_ent = entity_emb.shape
    ent, rel = pl.pallas_call(
        _tc_multi,
        in_specs=[
            pl.BlockSpec(memory_space=pltpu.MemorySpace.HBM),
            pl.BlockSpec(memory_space=pltpu.MemorySpace.HBM),
        ],
        out_specs=[
            pl.BlockSpec(memory_space=pltpu.MemorySpace.HBM),
            pl.BlockSpec(memory_space=pltpu.MemorySpace.HBM),
        ],
        out_shape=[
            jax.ShapeDtypeStruct(entity_emb.shape, entity_emb.dtype),
            jax.ShapeDtypeStruct(rel_emb.shape, rel_emb.dtype),
        ],
        scratch_shapes=[
            pltpu.VMEM((_NSTREAM, 2, _CHUNK, d_ent), entity_emb.dtype),
            pltpu.SemaphoreType.DMA((_NSTREAM, 2)),
            pltpu.SemaphoreType.DMA((_NSTREAM, 2)),
            pltpu.SemaphoreType.DMA,
        ],
    )(entity_emb, rel_emb)
    return (ent, rel)


# speedup vs baseline: 2.5902x; 1.0342x over previous
import jax
import jax.numpy as jnp
from jax.experimental import pallas as pl
from jax.experimental.pallas import tpu as pltpu

_BLK = 20000


def _copy_tables(ent_ref, rel_ref, ent_out, rel_out):
    ent_out[...] = ent_ref[...]

    @pl.when(pl.program_id(0) == 0)
    def _():
        rel_out[...] = rel_ref[...]


def kernel(x_dict, edge_index, entity_emb, rel_emb):
    del x_dict, edge_index
    n_ent, d_ent = entity_emb.shape
    n_rel, d_rel = rel_emb.shape
    ent, rel = pl.pallas_call(
        _copy_tables,
        grid=(n_ent // _BLK,),
        in_specs=[
            pl.BlockSpec((_BLK, d_ent), lambda i: (i, 0)),
            pl.BlockSpec((n_rel, d_rel), lambda i: (0, 0)),
        ],
        out_specs=[
            pl.BlockSpec((_BLK, d_ent), lambda i: (i, 0)),
            pl.BlockSpec((n_rel, d_rel), lambda i: (0, 0)),
        ],
        out_shape=[
            jax.ShapeDtypeStruct((n_ent, d_ent), entity_emb.dtype),
            jax.ShapeDtypeStruct((n_rel, d_rel), rel_emb.dtype),
        ],
    )(entity_emb, rel_emb)
    return (ent, rel)
